# baseline (speedup 1.0000x reference)
import functools

import jax
import jax.numpy as jnp
from jax import lax
from jax.experimental import pallas as pl
from jax.experimental.pallas import tpu as pltpu
from jax.experimental.pallas import tpu_sc as plsc

_B, _N = 16, 2048
_K = 16
_KM = 16
_LAYERS = [(3, 64, 1024), (64, 128, 256), (128, 256, 64), (256, 256, 16), (256, 512, 1)]
_BIG = 3.0e38


def _rb(x):
    return x.astype(jnp.bfloat16).astype(jnp.float32)


def _topk_body(npts, nprev, tile, pts_ref, ptst_ref, idx_ref):
    b = pl.program_id(0)
    t = pl.program_id(1)
    row0 = t * tile
    d2 = jnp.zeros((tile, nprev), jnp.float32)
    for c in range(3):
        sc = pts_ref[0, pl.ds(row0, tile), c:c + 1]
        pc = ptst_ref[0, c:c + 1, :nprev]
        diff = sc - pc
        d2 = d2 + diff * diff
    col = jax.lax.broadcasted_iota(jnp.int32, (tile, nprev), 1)
    vals = d2
    for k in range(_K):
        m = jnp.min(vals, axis=1, keepdims=True)
        cand = jnp.where(vals == m, col, nprev)
        loc = jnp.min(cand, axis=1, keepdims=True)
        idx_ref[0, :, k:k + 1] = loc + b * _N
        vals = jnp.where(col == loc, _BIG, vals)


def _topk_rounds(npts, nprev, d2, row_off, b, idx_ref):
    col = jax.lax.broadcasted_iota(jnp.int32, (npts, nprev), 1)
    vals = d2
    for k in range(_K):
        m = jnp.min(vals, axis=1, keepdims=True)
        cand = jnp.where(vals == m, col, nprev)
        loc = jnp.min(cand, axis=1, keepdims=True)
        idx_ref[0, :, k:k + 1] = loc + b * _N
        vals = jnp.where(col == loc, _BIG, vals)


def _topk_all_body(specs, pts_ref, ptst_ref, idx1_ref, *idx_refs):
    b = pl.program_id(0)
    t = pl.program_id(1)
    tile = 256
    row0 = t * tile
    d2 = jnp.zeros((tile, _N), jnp.float32)
    for c in range(3):
        sc = pts_ref[0, pl.ds(row0, tile), c:c + 1]
        pc = ptst_ref[0, c:c + 1, :]
        diff = sc - pc
        d2 = d2 + diff * diff
    _topk_rounds(tile, _N, d2, row0, b, idx1_ref)

    @pl.when(t == 3)
    def _():
        for (npts, nprev), idx_ref in zip(specs, idx_refs):
            d2 = jnp.zeros((npts, nprev), jnp.float32)
            for c in range(3):
                sc = pts_ref[0, 0:npts, c:c + 1]
                pc = ptst_ref[0, c:c + 1, :nprev]
                diff = sc - pc
                d2 = d2 + diff * diff
            _topk_rounds(npts, nprev, d2, 0, b, idx_ref)


def _make_topk_all(specs):
    return pl.pallas_call(
        functools.partial(_topk_all_body, specs),
        grid=(_B, 4),
        in_specs=[
            pl.BlockSpec((1, _N, 3), lambda b, t: (b, 0, 0)),
            pl.BlockSpec((1, 3, _N), lambda b, t: (b, 0, 0)),
        ],
        out_specs=[pl.BlockSpec((1, 256, _K), lambda b, t: (b, t, 0))] + [
            pl.BlockSpec((1, npts, _K), lambda b, t: (b, 0, 0))
            for npts, _ in specs],
        out_shape=[jax.ShapeDtypeStruct((_B, 1024, _K), jnp.int32)] + [
            jax.ShapeDtypeStruct((_B, npts, _K), jnp.int32)
            for npts, _ in specs],
    )


def _make_topk(npts, nprev):
    tile = min(npts, 256)
    return pl.pallas_call(
        functools.partial(_topk_body, npts, nprev, tile),
        grid=(_B, npts // tile),
        in_specs=[
            pl.BlockSpec((1, _N, 3), lambda b, t: (b, 0, 0)),
            pl.BlockSpec((1, 3, _N), lambda b, t: (b, 0, 0)),
        ],
        out_specs=pl.BlockSpec((1, tile, _K), lambda b, t: (b, t, 0)),
        out_shape=jax.ShapeDtypeStruct((_B, npts, _K), jnp.int32),
    )


def _conv_body(li, cin, cout, npts, cprev, count_prev, tile,
               nbrf_ref, nbrp_ref, pts_ref, wa_ref, ba_ref, w_ref, b_ref,
               gp_ref, bep_ref, sump_ref, sqp_ref,
               h_ref, sum_ref, sq_ref):
    b = pl.program_id(0)
    t = pl.program_id(1)
    nk = tile * _K
    nf = nbrf_ref[0][:, :cin]
    if li > 0:
        inv = 1.0 / count_prev
        mean = sump_ref[0:1, :] * inv
        var = sqp_ref[0:1, :] * inv - mean * mean
        s = gp_ref[0:1, :] * jax.lax.rsqrt(var + 1e-5)
        tt = bep_ref[0:1, :] - mean * s
        nf = jnp.maximum(nf * s + tt, 0.0)
    nfb = _rb(nf)
    np3 = nbrp_ref[0]
    sup = pts_ref[0, pl.ds(t * tile, tile), :]
    supr = jnp.broadcast_to(sup[:, None, :], (tile, _K, 3)).reshape(nk, 3)
    rel = np3[:, :3] - supr
    relb = _rb(rel)
    wab = _rb(wa_ref[...])
    a2 = jnp.zeros((nk, _KM), jnp.float32)
    for c in range(3):
        a2 = a2 + relb[:, c:c + 1] * wab[c:c + 1, :]
    a = jnp.maximum(a2 + ba_ref[0:1, :], 0.0)
    ab = _rb(a)
    acc = jnp.zeros((tile, cout), jnp.float32)
    if cin < 8:
        for c in range(cin):
            x = ab * nfb[:, c:c + 1]
            aggc = jnp.sum(x.reshape(tile, _K, _KM), axis=1) * (1.0 / _K)
            acc = acc + jnp.dot(aggc.astype(jnp.bfloat16),
                                w_ref[c * _KM:(c + 1) * _KM, :],
                                preferred_element_type=jnp.float32)
    else:
        a3 = ab.reshape(tile, _K, _KM)
        nf3 = nfb.reshape(tile, _K, cin)
        for m in range(_KM):
            aggm = jnp.sum(a3[:, :, m:m + 1] * nf3, axis=1) * (1.0 / _K)
            acc = acc + jnp.dot(aggm.astype(jnp.bfloat16),
                                w_ref[m * cin:(m + 1) * cin, :],
                                preferred_element_type=jnp.float32)
    h = acc + b_ref[0:1, :]
    h_ref[0, 0:tile, :] = h
    colsum = jnp.sum(h, axis=0, keepdims=True)
    colsq = jnp.sum(h * h, axis=0, keepdims=True)
    first = (b == 0) & (t == 0)

    @pl.when(first)
    def _():
        sum_ref[...] = colsum
        sq_ref[...] = colsq

    @pl.when(jnp.logical_not(first))
    def _():
        sum_ref[...] += colsum
        sq_ref[...] += colsq


def _make_conv(li, cin, cout, npts, cprev, count_prev, cf):
    tile = min(npts, 256)
    nk = tile * _K
    return pl.pallas_call(
        functools.partial(_conv_body, li, cin, cout, npts, cprev, count_prev,
                          tile),
        grid=(_B, npts // tile),
        in_specs=[
            pl.BlockSpec((1, nk, cf), lambda b, t: (b, t, 0)),
            pl.BlockSpec((1, nk, 16), lambda b, t: (b, t, 0)),
            pl.BlockSpec((1, _N, 3), lambda b, t: (b, 0, 0)),
            pl.BlockSpec((3, _KM), lambda b, t: (0, 0)),
            pl.BlockSpec((1, _KM), lambda b, t: (0, 0)),
            pl.BlockSpec((_KM * cin, cout), lambda b, t: (0, 0)),
            pl.BlockSpec((1, cout), lambda b, t: (0, 0)),
            pl.BlockSpec((1, cprev), lambda b, t: (0, 0)),
            pl.BlockSpec((1, cprev), lambda b, t: (0, 0)),
            pl.BlockSpec((1, cprev), lambda b, t: (0, 0)),
            pl.BlockSpec((1, cprev), lambda b, t: (0, 0)),
        ],
        out_specs=[
            pl.BlockSpec((1, max(tile, 8), cout), lambda b, t: (b, t, 0)),
            pl.BlockSpec((1, cout), lambda b, t: (0, 0)),
            pl.BlockSpec((1, cout), lambda b, t: (0, 0)),
        ],
        out_shape=[
            jax.ShapeDtypeStruct((_B, _N, cout), jnp.float32),
            jax.ShapeDtypeStruct((1, cout), jnp.float32),
            jax.ShapeDtypeStruct((1, cout), jnp.float32),
        ],
    )


def _fc_body(count, h_ref, sum_ref, sq_ref, g_ref, be_ref, wfc_ref, bfc_ref,
             out_ref):
    h = h_ref[:, :]
    inv = 1.0 / count
    mean = sum_ref[0:1, :] * inv
    var = sq_ref[0:1, :] * inv - mean * mean
    s = g_ref[0:1, :] * jax.lax.rsqrt(var + 1e-5)
    t = be_ref[0:1, :] - mean * s
    hn = jnp.maximum(h * s + t, 0.0)
    out_ref[:, :] = jnp.dot(hn.astype(jnp.bfloat16), wfc_ref[:, :],
                            preferred_element_type=jnp.float32) + bfc_ref[0:1, :]


def _make_sc_gather(total, df):
    nw = 32
    per_w = total // nw
    ch = min(per_w, 128)
    nch = per_w // ch
    mesh = plsc.VectorSubcoreMesh(core_axis_name="c", subcore_axis_name="s")

    @functools.partial(
        pl.kernel, mesh=mesh,
        out_type=[jax.ShapeDtypeStruct((total, df), jnp.float32),
                  jax.ShapeDtypeStruct((total, 16), jnp.float32)],
        scratch_types=[pltpu.VMEM((ch,), jnp.int32),
                       pltpu.VMEM((ch, df), jnp.float32),
                       pltpu.VMEM((ch, 16), jnp.float32),
                       pltpu.VMEM((ch,), jnp.int32),
                       pltpu.VMEM((ch, df), jnp.float32),
                       pltpu.VMEM((ch, 16), jnp.float32),
                       pltpu.SemaphoreType.DMA,
                       pltpu.SemaphoreType.DMA,
                       pltpu.SemaphoreType.DMA,
                       pltpu.SemaphoreType.DMA],
        compiler_params=pltpu.CompilerParams(use_tc_tiling_on_sc=False),
    )
    def gk(tf_hbm, tp_hbm, idx_hbm, outf_hbm, outp_hbm,
           idx_v0, rf_v0, rp_v0, idx_v1, rf_v1, rp_v1,
           semf0, semp0, semf1, semp1):
        wid = lax.axis_index("s") * 2 + lax.axis_index("c")
        base = wid * per_w
        buf0 = (idx_v0, rf_v0, rp_v0, semf0, semp0)
        buf1 = (idx_v1, rf_v1, rp_v1, semf1, semp1)

        def fire(buf, c):
            idx_v, rf_v, rp_v, semf, semp = buf
            off = jnp.minimum(base + c * ch, total - ch)
            pltpu.sync_copy(idx_hbm.at[pl.ds(off, ch)], idx_v)
            pltpu.async_copy(tf_hbm.at[idx_v], rf_v, semf)
            pltpu.async_copy(tp_hbm.at[idx_v], rp_v, semp)

        def drain(buf):
            idx_v, rf_v, rp_v, semf, semp = buf
            pltpu.make_async_copy(tf_hbm.at[idx_v], rf_v, semf).wait()
            pltpu.make_async_copy(tp_hbm.at[idx_v], rp_v, semp).wait()

        def store(buf, c):
            idx_v, rf_v, rp_v, semf, semp = buf
            off = base + c * ch
            pltpu.sync_copy(rf_v, outf_hbm.at[pl.ds(off, ch)])
            pltpu.sync_copy(rp_v, outp_hbm.at[pl.ds(off, ch)])

        if nch == 1:
            fire(buf0, 0)
            drain(buf0)
            store(buf0, 0)
        else:
            fire(buf0, 0)

            def body(p, carry):
                c0 = p * 2
                fire(buf1, c0 + 1)
                drain(buf0)
                store(buf0, c0)
                fire(buf0, c0 + 2)
                drain(buf1)
                store(buf1, c0 + 1)
                return carry

            lax.fori_loop(0, nch // 2, body, 0)
            drain(buf0)

    return gk


def _gather_rows(tf, tp, idx):
    total = idx.size
    return _make_sc_gather(total, tf.shape[1])(tf, tp, idx.reshape(-1))


def kernel(x, input_pts, Wa1, ba1, W1, b1, g1, be1, Wa2, ba2, W2, b2, g2, be2, Wa3, ba3, W3, b3, g3, be3, Wa4, ba4, W4, b4, g4, be4, Wa5, ba5, W5, b5, g5, be5, Wfc, bfc):
    params = [Wa1, ba1, W1, b1, g1, be1, Wa2, ba2, W2, b2, g2, be2, Wa3, ba3, W3, b3, g3, be3, Wa4, ba4, W4, b4, g4, be4, Wa5, ba5, W5, b5, g5, be5]
    ptst = jnp.transpose(input_pts, (0, 2, 1))
    tp = jnp.pad(input_pts, ((0, 0), (0, 0), (0, 13))).reshape(_B * _N, 16)

    h_raw = jnp.pad(x, ((0, 0), (0, 0), (0, 13))).reshape(_B * _N, 16)
    cf = 16
    cprev, count_prev = 1, 1.0
    gp = bep = sump = sqp = jnp.zeros((1, 1), jnp.float32)
    nprev = _N
    idxs = list(_make_topk_all([(256, 1024), (64, 256), (16, 64), (1, 16)])(
        input_pts, ptst))
    for li, (cin, cout, npts) in enumerate(_LAYERS):
        Wa, ba, W, b, g, be = params[6 * li:6 * li + 6]
        if cin < 8:
            W = W.reshape(_KM, cin, cout).transpose(1, 0, 2).reshape(cin * _KM, cout)
        W = W.astype(jnp.bfloat16)
        idx = idxs[li]
        nbrf, nbrp = _gather_rows(h_raw, tp, idx)
        nbrf = nbrf.reshape(_B, npts * _K, cf)
        nbrp = nbrp.reshape(_B, npts * _K, 16)
        h_new, hsum, hsq = _make_conv(li, cin, cout, npts, cprev, count_prev, cf)(
            nbrf, nbrp, input_pts, Wa, ba.reshape(1, _KM), W, b.reshape(1, cout),
            gp, bep, sump, sqp)
        h_raw = h_new.reshape(_B * _N, cout)
        cf = cout
        cprev, count_prev = cout, float(_B * npts)
        gp, bep = g.reshape(1, cout), be.reshape(1, cout)
        sump, sqp = hsum, hsq
        nprev = npts

    h5 = h_raw.reshape(_B, _N, 512)[:, 0, :]
    out = pl.pallas_call(
        _fc_body_bound := functools.partial(_fc_body, count_prev),
        out_shape=jax.ShapeDtypeStruct((_B, 40), jnp.float32),
    )(h5, sump, sqp, gp, bep, Wfc.astype(jnp.bfloat16), bfc.reshape(1, 40))
    return out


# speedup vs baseline: 1.0304x; 1.0304x over previous
import functools

import jax
import jax.numpy as jnp
from jax import lax
from jax.experimental import pallas as pl
from jax.experimental.pallas import tpu as pltpu
from jax.experimental.pallas import tpu_sc as plsc

_B, _N = 16, 2048
_K = 16
_KM = 16
_LAYERS = [(3, 64, 1024), (64, 128, 256), (128, 256, 64), (256, 256, 16), (256, 512, 1)]
_BIG = 3.0e38


def _rb(x):
    return x.astype(jnp.bfloat16).astype(jnp.float32)


def _topk_body(npts, nprev, tile, pts_ref, ptst_ref, idx_ref):
    b = pl.program_id(0)
    t = pl.program_id(1)
    row0 = t * tile
    d2 = jnp.zeros((tile, nprev), jnp.float32)
    for c in range(3):
        sc = pts_ref[0, pl.ds(row0, tile), c:c + 1]
        pc = ptst_ref[0, c:c + 1, :nprev]
        diff = sc - pc
        d2 = d2 + diff * diff
    col = jax.lax.broadcasted_iota(jnp.int32, (tile, nprev), 1)
    vals = d2
    for k in range(_K):
        m = jnp.min(vals, axis=1, keepdims=True)
        cand = jnp.where(vals == m, col, nprev)
        loc = jnp.min(cand, axis=1, keepdims=True)
        idx_ref[0, :, k:k + 1] = loc + b * _N
        vals = jnp.where(col == loc, _BIG, vals)


def _topk_rounds(npts, nprev, d2, row_off, b, idx_ref):
    col = jax.lax.broadcasted_iota(jnp.int32, (npts, nprev), 1)
    vals = d2
    for k in range(_K):
        m = jnp.min(vals, axis=1, keepdims=True)
        cand = jnp.where(vals == m, col, nprev)
        loc = jnp.min(cand, axis=1, keepdims=True)
        idx_ref[0, :, k:k + 1] = loc + b * _N
        vals = jnp.where(col == loc, _BIG, vals)


def _topk4_body(specs, pts_ref, ptst_ref, *idx_refs):
    b = pl.program_id(0)
    for (npts, nprev), idx_ref in zip(specs, idx_refs):
        d2 = jnp.zeros((npts, nprev), jnp.float32)
        for c in range(3):
            sc = pts_ref[0, 0:npts, c:c + 1]
            pc = ptst_ref[0, c:c + 1, :nprev]
            diff = sc - pc
            d2 = d2 + diff * diff
        _topk_rounds(npts, nprev, d2, 0, b, idx_ref)


def _make_topk4(specs):
    return pl.pallas_call(
        functools.partial(_topk4_body, specs),
        grid=(_B,),
        in_specs=[
            pl.BlockSpec((1, _N, 3), lambda b: (b, 0, 0)),
            pl.BlockSpec((1, 3, _N), lambda b: (b, 0, 0)),
        ],
        out_specs=[pl.BlockSpec((1, npts, _K), lambda b: (b, 0, 0))
                   for npts, _ in specs],
        out_shape=[jax.ShapeDtypeStruct((_B, npts, _K), jnp.int32)
                   for npts, _ in specs],
    )


def _make_topk(npts, nprev):
    tile = min(npts, 256)
    return pl.pallas_call(
        functools.partial(_topk_body, npts, nprev, tile),
        grid=(_B, npts // tile),
        in_specs=[
            pl.BlockSpec((1, _N, 3), lambda b, t: (b, 0, 0)),
            pl.BlockSpec((1, 3, _N), lambda b, t: (b, 0, 0)),
        ],
        out_specs=pl.BlockSpec((1, tile, _K), lambda b, t: (b, t, 0)),
        out_shape=jax.ShapeDtypeStruct((_B, npts, _K), jnp.int32),
    )


def _conv_body(li, cin, cout, npts, cprev, count_prev, tile,
               nbrf_ref, nbrp_ref, pts_ref, wa_ref, ba_ref, w_ref, b_ref,
               gp_ref, bep_ref, sump_ref, sqp_ref,
               h_ref, sum_ref, sq_ref):
    b = pl.program_id(0)
    t = pl.program_id(1)
    nk = tile * _K
    nf = nbrf_ref[0][:, :cin]
    if li > 0:
        inv = 1.0 / count_prev
        mean = sump_ref[0:1, :] * inv
        var = sqp_ref[0:1, :] * inv - mean * mean
        s = gp_ref[0:1, :] * jax.lax.rsqrt(var + 1e-5)
        tt = bep_ref[0:1, :] - mean * s
        nf = jnp.maximum(nf * s + tt, 0.0)
    nfb = _rb(nf)
    np3 = nbrp_ref[0]
    sup = pts_ref[0, pl.ds(t * tile, tile), :]
    supr = jnp.broadcast_to(sup[:, None, :], (tile, _K, 3)).reshape(nk, 3)
    rel = np3[:, :3] - supr
    relb = _rb(rel)
    wab = _rb(wa_ref[...])
    a2 = jnp.zeros((nk, _KM), jnp.float32)
    for c in range(3):
        a2 = a2 + relb[:, c:c + 1] * wab[c:c + 1, :]
    a = jnp.maximum(a2 + ba_ref[0:1, :], 0.0)
    ab = _rb(a)
    acc = jnp.zeros((tile, cout), jnp.float32)
    if cin < 8:
        for c in range(cin):
            x = ab * nfb[:, c:c + 1]
            aggc = jnp.sum(x.reshape(tile, _K, _KM), axis=1) * (1.0 / _K)
            acc = acc + jnp.dot(aggc.astype(jnp.bfloat16),
                                w_ref[c * _KM:(c + 1) * _KM, :],
                                preferred_element_type=jnp.float32)
    else:
        a3 = ab.reshape(tile, _K, _KM)
        nf3 = nfb.reshape(tile, _K, cin)
        for m in range(_KM):
            aggm = jnp.sum(a3[:, :, m:m + 1] * nf3, axis=1) * (1.0 / _K)
            acc = acc + jnp.dot(aggm.astype(jnp.bfloat16),
                                w_ref[m * cin:(m + 1) * cin, :],
                                preferred_element_type=jnp.float32)
    h = acc + b_ref[0:1, :]
    h_ref[0, 0:tile, :] = h
    colsum = jnp.sum(h, axis=0, keepdims=True)
    colsq = jnp.sum(h * h, axis=0, keepdims=True)
    first = (b == 0) & (t == 0)

    @pl.when(first)
    def _():
        sum_ref[...] = colsum
        sq_ref[...] = colsq

    @pl.when(jnp.logical_not(first))
    def _():
        sum_ref[...] += colsum
        sq_ref[...] += colsq


def _make_conv(li, cin, cout, npts, cprev, count_prev, cf):
    tile = min(npts, 256)
    nk = tile * _K
    return pl.pallas_call(
        functools.partial(_conv_body, li, cin, cout, npts, cprev, count_prev,
                          tile),
        grid=(_B, npts // tile),
        in_specs=[
            pl.BlockSpec((1, nk, cf), lambda b, t: (b, t, 0)),
            pl.BlockSpec((1, nk, 16), lambda b, t: (b, t, 0)),
            pl.BlockSpec((1, _N, 3), lambda b, t: (b, 0, 0)),
            pl.BlockSpec((3, _KM), lambda b, t: (0, 0)),
            pl.BlockSpec((1, _KM), lambda b, t: (0, 0)),
            pl.BlockSpec((_KM * cin, cout), lambda b, t: (0, 0)),
            pl.BlockSpec((1, cout), lambda b, t: (0, 0)),
            pl.BlockSpec((1, cprev), lambda b, t: (0, 0)),
            pl.BlockSpec((1, cprev), lambda b, t: (0, 0)),
            pl.BlockSpec((1, cprev), lambda b, t: (0, 0)),
            pl.BlockSpec((1, cprev), lambda b, t: (0, 0)),
        ],
        out_specs=[
            pl.BlockSpec((1, max(tile, 8), cout), lambda b, t: (b, t, 0)),
            pl.BlockSpec((1, cout), lambda b, t: (0, 0)),
            pl.BlockSpec((1, cout), lambda b, t: (0, 0)),
        ],
        out_shape=[
            jax.ShapeDtypeStruct((_B, _N, cout), jnp.float32),
            jax.ShapeDtypeStruct((1, cout), jnp.float32),
            jax.ShapeDtypeStruct((1, cout), jnp.float32),
        ],
    )


def _fc_body(count, h_ref, sum_ref, sq_ref, g_ref, be_ref, wfc_ref, bfc_ref,
             out_ref):
    h = h_ref[:, :]
    inv = 1.0 / count
    mean = sum_ref[0:1, :] * inv
    var = sq_ref[0:1, :] * inv - mean * mean
    s = g_ref[0:1, :] * jax.lax.rsqrt(var + 1e-5)
    t = be_ref[0:1, :] - mean * s
    hn = jnp.maximum(h * s + t, 0.0)
    out_ref[:, :] = jnp.dot(hn.astype(jnp.bfloat16), wfc_ref[:, :],
                            preferred_element_type=jnp.float32) + bfc_ref[0:1, :]


def _make_sc_gather(total, df):
    nw = 32
    per_w = total // nw
    ch = min(per_w, 128)
    nch = per_w // ch
    mesh = plsc.VectorSubcoreMesh(core_axis_name="c", subcore_axis_name="s")

    @functools.partial(
        pl.kernel, mesh=mesh,
        out_type=[jax.ShapeDtypeStruct((total, df), jnp.float32),
                  jax.ShapeDtypeStruct((total, 16), jnp.float32)],
        scratch_types=[pltpu.VMEM((ch,), jnp.int32),
                       pltpu.VMEM((ch, df), jnp.float32),
                       pltpu.VMEM((ch, 16), jnp.float32),
                       pltpu.VMEM((ch,), jnp.int32),
                       pltpu.VMEM((ch, df), jnp.float32),
                       pltpu.VMEM((ch, 16), jnp.float32),
                       pltpu.SemaphoreType.DMA,
                       pltpu.SemaphoreType.DMA,
                       pltpu.SemaphoreType.DMA,
                       pltpu.SemaphoreType.DMA],
        compiler_params=pltpu.CompilerParams(use_tc_tiling_on_sc=False),
    )
    def gk(tf_hbm, tp_hbm, idx_hbm, outf_hbm, outp_hbm,
           idx_v0, rf_v0, rp_v0, idx_v1, rf_v1, rp_v1,
           semf0, semp0, semf1, semp1):
        wid = lax.axis_index("s") * 2 + lax.axis_index("c")
        base = wid * per_w
        buf0 = (idx_v0, rf_v0, rp_v0, semf0, semp0)
        buf1 = (idx_v1, rf_v1, rp_v1, semf1, semp1)

        def fire(buf, c):
            idx_v, rf_v, rp_v, semf, semp = buf
            off = jnp.minimum(base + c * ch, total - ch)
            pltpu.sync_copy(idx_hbm.at[pl.ds(off, ch)], idx_v)
            pltpu.async_copy(tf_hbm.at[idx_v], rf_v, semf)
            pltpu.async_copy(tp_hbm.at[idx_v], rp_v, semp)

        def drain(buf):
            idx_v, rf_v, rp_v, semf, semp = buf
            pltpu.make_async_copy(tf_hbm.at[idx_v], rf_v, semf).wait()
            pltpu.make_async_copy(tp_hbm.at[idx_v], rp_v, semp).wait()

        def store(buf, c):
            idx_v, rf_v, rp_v, semf, semp = buf
            off = base + c * ch
            pltpu.sync_copy(rf_v, outf_hbm.at[pl.ds(off, ch)])
            pltpu.sync_copy(rp_v, outp_hbm.at[pl.ds(off, ch)])

        if nch == 1:
            fire(buf0, 0)
            drain(buf0)
            store(buf0, 0)
        else:
            fire(buf0, 0)

            def body(p, carry):
                c0 = p * 2
                fire(buf1, c0 + 1)
                drain(buf0)
                store(buf0, c0)
                fire(buf0, c0 + 2)
                drain(buf1)
                store(buf1, c0 + 1)
                return carry

            lax.fori_loop(0, nch // 2, body, 0)
            drain(buf0)

    return gk


def _gather_rows(tf, tp, idx):
    total = idx.size
    return _make_sc_gather(total, tf.shape[1])(tf, tp, idx.reshape(-1))


def kernel(x, input_pts, Wa1, ba1, W1, b1, g1, be1, Wa2, ba2, W2, b2, g2, be2, Wa3, ba3, W3, b3, g3, be3, Wa4, ba4, W4, b4, g4, be4, Wa5, ba5, W5, b5, g5, be5, Wfc, bfc):
    params = [Wa1, ba1, W1, b1, g1, be1, Wa2, ba2, W2, b2, g2, be2, Wa3, ba3, W3, b3, g3, be3, Wa4, ba4, W4, b4, g4, be4, Wa5, ba5, W5, b5, g5, be5]
    ptst = jnp.transpose(input_pts, (0, 2, 1))
    tp = jnp.pad(input_pts, ((0, 0), (0, 0), (0, 13))).reshape(_B * _N, 16)

    h_raw = jnp.pad(x, ((0, 0), (0, 0), (0, 13))).reshape(_B * _N, 16)
    cf = 16
    cprev, count_prev = 1, 1.0
    gp = bep = sump = sqp = jnp.zeros((1, 1), jnp.float32)
    nprev = _N
    idx1 = _make_topk(1024, _N)(input_pts, ptst)
    idx_rest = _make_topk4([(256, 1024), (64, 256), (16, 64), (1, 16)])(
        input_pts, ptst)
    idxs = [idx1] + list(idx_rest)
    for li, (cin, cout, npts) in enumerate(_LAYERS):
        Wa, ba, W, b, g, be = params[6 * li:6 * li + 6]
        if cin < 8:
            W = W.reshape(_KM, cin, cout).transpose(1, 0, 2).reshape(cin * _KM, cout)
        W = W.astype(jnp.bfloat16)
        idx = idxs[li]
        nbrf, nbrp = _gather_rows(h_raw, tp, idx)
        nbrf = nbrf.reshape(_B, npts * _K, cf)
        nbrp = nbrp.reshape(_B, npts * _K, 16)
        h_new, hsum, hsq = _make_conv(li, cin, cout, npts, cprev, count_prev, cf)(
            nbrf, nbrp, input_pts, Wa, ba.reshape(1, _KM), W, b.reshape(1, cout),
            gp, bep, sump, sqp)
        h_raw = h_new.reshape(_B * _N, cout)
        cf = cout
        cprev, count_prev = cout, float(_B * npts)
        gp, bep = g.reshape(1, cout), be.reshape(1, cout)
        sump, sqp = hsum, hsq
        nprev = npts

    h5 = h_raw.reshape(_B, _N, 512)[:, 0, :]
    out = pl.pallas_call(
        _fc_body_bound := functools.partial(_fc_body, count_prev),
        out_shape=jax.ShapeDtypeStruct((_B, 40), jnp.float32),
    )(h5, sump, sqp, gp, bep, Wfc.astype(jnp.bfloat16), bfc.reshape(1, 40))
    return out
